# baseline (speedup 1.0000x reference)
import functools

import jax
import jax.numpy as jnp
from jax import lax
from jax.experimental import pallas as pl
from jax.experimental.pallas import tpu as pltpu
from jax.experimental.pallas import tpu_sc as plsc

N = 10000
E = 320000
H = 128
EH = 256
G = 128

NC = 2
NS = 16
NW = NC * NS
EW = E // NW
CH = 40
NCH = EW // CH
NG = 5
CPG = NCH // NG
NB = 3
ZS = 10
ZRW = N // ZS

TN = 2000
TE = 4000

_f32 = jnp.float32


def _dot(a, b):
    return jnp.dot(a, b, preferred_element_type=_f32)


def _dot_hi(a, b):
    return jnp.dot(a, b, preferred_element_type=_f32,
                   precision=lax.Precision.HIGHEST)


_BN_DEN = 1.0000050067901611


def _bnk(x, g, be):
    return g * x / _BN_DEN + be


def _onehot(bt):
    iota = lax.broadcasted_iota(jnp.int32, (bt.shape[0], G), 1)
    return jnp.where(bt == iota, 1.0, 0.0).astype(_f32)


def _in_body(x_ref, w_ref, b_ref, v_ref, o_ref):
    h = jnp.maximum(_dot(x_ref[...], w_ref[...]) + b_ref[...], 0.0)
    o_ref[...] = h + v_ref[...]


def _input_layer(x, atom_w, atom_b, vn_row):
    return pl.pallas_call(
        _in_body,
        grid=(N // TN,),
        in_specs=[
            pl.BlockSpec((TN, H), lambda i: (i, 0)),
            pl.BlockSpec((H, H), lambda i: (0, 0)),
            pl.BlockSpec((1, H), lambda i: (0, 0)),
            pl.BlockSpec((1, H), lambda i: (0, 0)),
        ],
        out_specs=pl.BlockSpec((TN, H), lambda i: (i, 0)),
        out_shape=jax.ShapeDtypeStruct((N, H), _f32),
    )(x, atom_w, atom_b, vn_row)


def _edge_body(ea_ref, *refs):
    ea = ea_ref[...]
    for l in range(3):
        w1, b1, w2, b2 = refs[4 * l:4 * l + 4]
        t = jnp.maximum(_dot(ea, w1[...]) + b1[...], 0.0)
        refs[12 + l][...] = _dot(t, w2[...]) + b2[...]


def _edge_mlps(edge_attr, ws):
    de = edge_attr.shape[1]
    wspecs = []
    for _ in range(3):
        wspecs += [
            pl.BlockSpec((de, EH), lambda i: (0, 0)),
            pl.BlockSpec((1, EH), lambda i: (0, 0)),
            pl.BlockSpec((EH, H), lambda i: (0, 0)),
            pl.BlockSpec((1, H), lambda i: (0, 0)),
        ]
    return pl.pallas_call(
        _edge_body,
        grid=(E // TE,),
        in_specs=[pl.BlockSpec((TE, de), lambda i: (i, 0))] + wspecs,
        out_specs=[pl.BlockSpec((TE, H), lambda i: (i, 0))] * 3,
        out_shape=[jax.ShapeDtypeStruct((E, H), _f32)] * 3,
    )(edge_attr, *ws)


def _vn_body(h_ref, bf_ref, vn_ref, w1, b1, g1, be1, w2, b2, g2, be2,
             o_ref, acc):
    i = pl.program_id(0)

    @pl.when(i == 0)
    def _():
        acc[...] = jnp.zeros_like(acc)

    oh = _onehot(bf_ref[...])
    acc[...] += lax.dot_general(oh, h_ref[...], (((0,), (0,)), ((), ())),
                                preferred_element_type=_f32,
                                precision=lax.Precision.HIGHEST)

    @pl.when(i == pl.num_programs(0) - 1)
    def _():
        vt = acc[...] + vn_ref[...]
        t2 = jnp.maximum(_bnk(_dot(vt, w1[...]) + b1[...], g1[...], be1[...]),
                         0.0)
        o_ref[...] = jnp.maximum(
            _bnk(_dot(t2, w2[...]) + b2[...], g2[...], be2[...]), 0.0)


def _vn_update(h_cur, batch_f, vn, v):
    return pl.pallas_call(
        _vn_body,
        grid=(N // TN,),
        in_specs=[
            pl.BlockSpec((TN, H), lambda i: (i, 0)),
            pl.BlockSpec((TN, 1), lambda i: (i, 0)),
            pl.BlockSpec((G, H), lambda i: (0, 0)),
            pl.BlockSpec((H, EH), lambda i: (0, 0)),
            pl.BlockSpec((1, EH), lambda i: (0, 0)),
            pl.BlockSpec((1, EH), lambda i: (0, 0)),
            pl.BlockSpec((1, EH), lambda i: (0, 0)),
            pl.BlockSpec((EH, H), lambda i: (0, 0)),
            pl.BlockSpec((1, H), lambda i: (0, 0)),
            pl.BlockSpec((1, H), lambda i: (0, 0)),
            pl.BlockSpec((1, H), lambda i: (0, 0)),
        ],
        out_specs=pl.BlockSpec((G, H), lambda i: (0, 0)),
        out_shape=jax.ShapeDtypeStruct((G, H), _f32),
        scratch_shapes=[pltpu.VMEM((G, H), _f32)],
    )(h_cur, batch_f, vn,
      v["W1"], v["b1"].reshape(1, EH), v["g1"].reshape(1, EH),
      v["be1"].reshape(1, EH),
      v["W2"], v["b2"].reshape(1, H), v["g2"].reshape(1, H),
      v["be2"].reshape(1, H))


@functools.lru_cache(maxsize=None)
def _build_sc_aggregate():
    mesh = plsc.VectorSubcoreMesh(core_axis_name="c", subcore_axis_name="s",
                                  num_cores=NC, num_subcores=NS)
    return functools.partial(
        pl.kernel,
        out_type=jax.ShapeDtypeStruct((2 * N, H), _f32),
        mesh=mesh,
        scratch_types=[
            pltpu.VMEM((CPG, CH), jnp.int32),
            pltpu.VMEM((CPG, CH), jnp.int32),
            [pltpu.VMEM((CH, H), _f32)] * NB,
            [pltpu.VMEM((CH, H), _f32)] * NB,
            pltpu.VMEM_SHARED((N, H), _f32),
            [pltpu.SemaphoreType.DMA] * NB,
            [pltpu.SemaphoreType.DMA] * NB,
        ],
    )(_sc_body)


def _sc_body(h_hbm, ee_hbm, src_hbm, dst_hbm, z_hbm, out_hbm,
             src_v, dst_v, ee_b, hr_b, acc_sh, sem_l, sem_s):
    c = lax.axis_index("c")
    s = lax.axis_index("s")
    wid = c * NS + s

    @pl.when(s < ZS)
    def _():
        pltpu.sync_copy(z_hbm, acc_sh.at[pl.ds(s * ZRW, ZRW)])

    plsc.subcore_barrier()

    base0 = wid * EW

    def issue_loads(g, i, k):
        base = base0 + g * CPG * CH + i * CH
        pltpu.async_copy(ee_hbm.at[pl.ds(base, CH)], ee_b[k], sem_l[k])
        pltpu.async_copy(h_hbm.at[src_v.at[i]], hr_b[k], sem_l[k])

    def wait_loads(g, i, k):
        base = base0 + g * CPG * CH + i * CH
        pltpu.make_async_copy(ee_hbm.at[pl.ds(base, CH)], ee_b[k],
                              sem_l[k]).wait()
        pltpu.make_async_copy(h_hbm.at[src_v.at[i]], hr_b[k],
                              sem_l[k]).wait()

    def wait_scatter(i, k):
        pltpu.make_async_copy(hr_b[k], acc_sh.at[dst_v.at[i]],
                              sem_s[k]).wait()

    def slot(g, i, k, guard=True, issue=True):
        wait_loads(g, i, k)
        kk = (k + 1) % NB

        if guard:
            @pl.when(i >= 2)
            def _():
                wait_scatter(i - 2, kk)
        else:
            wait_scatter(i - 2, kk)

        if issue:
            issue_loads(g, i + 1, kk)

        @plsc.parallel_loop(0, CH, unroll=4)
        def _(r):
            for j in range(H // 16):
                sl = pl.ds(j * 16, 16)
                hr_b[k][r, sl] = jnp.maximum(
                    hr_b[k][r, sl] + ee_b[k][r, sl], 0.0)

        pltpu.async_copy(hr_b[k], acc_sh.at[dst_v.at[i]], sem_s[k], add=True)

    def group(g, carry):
        pltpu.sync_copy(src_hbm.at[wid, g], src_v)
        pltpu.sync_copy(dst_hbm.at[wid, g], dst_v)
        issue_loads(g, 0, 0)

        def macro(m, carry1):
            for j in range(NB):
                slot(g, m * NB + j, j)
            return carry1

        lax.fori_loop(0, (CPG - 2) // NB, macro, 0)
        slot(g, CPG - 2, (CPG - 2) % NB, guard=False, issue=True)
        slot(g, CPG - 1, (CPG - 1) % NB, guard=False, issue=False)
        wait_scatter(CPG - 2, (CPG - 2) % NB)
        wait_scatter(CPG - 1, (CPG - 1) % NB)
        return carry

    lax.fori_loop(0, NG, group, 0)
    plsc.subcore_barrier()

    @pl.when(s < ZS)
    def _():
        pltpu.sync_copy(acc_sh.at[pl.ds(s * ZRW, ZRW)],
                        out_hbm.at[pl.ds(c * N + s * ZRW, ZRW)])


def _edge_aggregate(h_cur, ee, src2, dst2, zrows):
    return _build_sc_aggregate()(h_cur, ee, src2, dst2, zrows)


def _tail_body(h_ref, a0_ref, a1_ref, bf_ref, vnn_ref, w1, b1, g1, be1,
               w2, b2, ep_ref, o_ref):
    t = ep_ref[0, 0] * h_ref[...] + a0_ref[...] + a1_ref[...]
    u = jnp.maximum(_bnk(_dot(t, w1[...]) + b1[...], g1[...], be1[...]), 0.0)
    hn = jnp.maximum(_dot(u, w2[...]) + b2[...], 0.0)
    o_ref[...] = hn + _dot_hi(_onehot(bf_ref[...]), vnn_ref[...])


def _tail(h_cur, aggr2, batch_f, vn_next, c, ep):
    nt = N // TN
    return pl.pallas_call(
        _tail_body,
        grid=(nt,),
        in_specs=[
            pl.BlockSpec((TN, H), lambda i: (i, 0)),
            pl.BlockSpec((TN, H), lambda i: (i, 0)),
            pl.BlockSpec((TN, H), lambda i, nt=nt: (i + nt, 0)),
            pl.BlockSpec((TN, 1), lambda i: (i, 0)),
            pl.BlockSpec((G, H), lambda i: (0, 0)),
            pl.BlockSpec((H, EH), lambda i: (0, 0)),
            pl.BlockSpec((1, EH), lambda i: (0, 0)),
            pl.BlockSpec((1, EH), lambda i: (0, 0)),
            pl.BlockSpec((1, EH), lambda i: (0, 0)),
            pl.BlockSpec((EH, H), lambda i: (0, 0)),
            pl.BlockSpec((1, H), lambda i: (0, 0)),
            pl.BlockSpec((1, 1), lambda i: (0, 0)),
        ],
        out_specs=pl.BlockSpec((TN, H), lambda i: (i, 0)),
        out_shape=jax.ShapeDtypeStruct((N, H), _f32),
    )(h_cur, aggr2, aggr2, batch_f, vn_next,
      c["mW1"], c["mb1"].reshape(1, EH), c["mg1"].reshape(1, EH),
      c["mbe1"].reshape(1, EH),
      c["mW2"], c["mb2"].reshape(1, H), ep)


def _final_body(h_ref, a0_ref, a1_ref, bf_ref, w1, b1, g1, be1, w2, b2,
                ep_ref, l1, c1, l2, c2, l3, c3, l4, c4, o_ref, pool, cnt):
    i = pl.program_id(0)

    @pl.when(i == 0)
    def _():
        pool[...] = jnp.zeros_like(pool)
        cnt[...] = jnp.zeros_like(cnt)

    t = ep_ref[0, 0] * h_ref[...] + a0_ref[...] + a1_ref[...]
    u = jnp.maximum(_bnk(_dot(t, w1[...]) + b1[...], g1[...], be1[...]), 0.0)
    h3 = _dot(u, w2[...]) + b2[...]
    oh = _onehot(bf_ref[...])
    dn = (((0,), (0,)), ((), ()))
    hp = lax.Precision.HIGHEST
    pool[...] += lax.dot_general(oh, h3, dn, preferred_element_type=_f32,
                                 precision=hp)
    cnt[...] += lax.dot_general(oh, jnp.ones((oh.shape[0], 1), _f32), dn,
                                preferred_element_type=_f32, precision=hp)

    @pl.when(i == pl.num_programs(0) - 1)
    def _():
        hg = pool[...] / jnp.maximum(cnt[...], 1.0)
        o = _dot(hg, l1[...]) + c1[...]
        o = _dot(o, l2[...]) + c2[...]
        o = _dot(o, l3[...]) + c3[...]
        o_ref[...] = _dot(o, l4[...]) + c4[...]


def _final(h_cur, aggr2, batch_f, c, ep, lin):
    nt = N // TN
    return pl.pallas_call(
        _final_body,
        grid=(nt,),
        in_specs=[
            pl.BlockSpec((TN, H), lambda i: (i, 0)),
            pl.BlockSpec((TN, H), lambda i: (i, 0)),
            pl.BlockSpec((TN, H), lambda i, nt=nt: (i + nt, 0)),
            pl.BlockSpec((TN, 1), lambda i: (i, 0)),
            pl.BlockSpec((H, EH), lambda i: (0, 0)),
            pl.BlockSpec((1, EH), lambda i: (0, 0)),
            pl.BlockSpec((1, EH), lambda i: (0, 0)),
            pl.BlockSpec((1, EH), lambda i: (0, 0)),
            pl.BlockSpec((EH, H), lambda i: (0, 0)),
            pl.BlockSpec((1, H), lambda i: (0, 0)),
            pl.BlockSpec((1, 1), lambda i: (0, 0)),
            pl.BlockSpec((H, H), lambda i: (0, 0)),
            pl.BlockSpec((1, H), lambda i: (0, 0)),
            pl.BlockSpec((H, H), lambda i: (0, 0)),
            pl.BlockSpec((1, H), lambda i: (0, 0)),
            pl.BlockSpec((H, H), lambda i: (0, 0)),
            pl.BlockSpec((1, H), lambda i: (0, 0)),
            pl.BlockSpec((H, 1), lambda i: (0, 0)),
            pl.BlockSpec((1, 1), lambda i: (0, 0)),
        ],
        out_specs=pl.BlockSpec((G, 1), lambda i: (0, 0)),
        out_shape=jax.ShapeDtypeStruct((G, 1), _f32),
        scratch_shapes=[pltpu.VMEM((G, H), _f32), pltpu.VMEM((G, 1), _f32)],
    )(h_cur, aggr2, aggr2, batch_f,
      c["mW1"], c["mb1"].reshape(1, EH), c["mg1"].reshape(1, EH),
      c["mbe1"].reshape(1, EH),
      c["mW2"], c["mb2"].reshape(1, H), ep,
      lin["W1"], lin["b1"].reshape(1, H),
      lin["W2"], lin["b2"].reshape(1, H),
      lin["W3"], lin["b3"].reshape(1, H),
      lin["W4"], lin["b4"].reshape(1, 1))


def kernel(x, edge_index, edge_attr, batch, params):
    batch_f = batch.reshape(N, 1)
    src2 = edge_index[0].reshape(NW, NG, CPG, CH)
    dst2 = edge_index[1].reshape(NW, NG, CPG, CH)
    zrows = jnp.zeros((ZRW, H), _f32)

    vn_row = params["vn_emb"]
    h_cur = _input_layer(x, params["atom_W"], params["atom_b"].reshape(1, H),
                         vn_row)

    ews = []
    for l in range(3):
        c = params["convs"][l]
        ews += [c["eW1"], c["eb1"].reshape(1, EH), c["eW2"],
                c["eb2"].reshape(1, H)]
    ee0, ee1, ee2 = _edge_mlps(edge_attr, ews)
    ees = [ee0, ee1, ee2]

    vn = jnp.broadcast_to(vn_row, (G, H))
    for l in range(3):
        c = params["convs"][l]
        ep = (1.0 + c["eps"]).reshape(1, 1)

        aggr2 = _edge_aggregate(h_cur, ees[l], src2, dst2, zrows)

        if l < 2:
            vn = _vn_update(h_cur, batch_f, vn, params["vns"][l])
            h_cur = _tail(h_cur, aggr2, batch_f, vn, c, ep)
        else:
            o = _final(h_cur, aggr2, batch_f, c, ep, params["lin"])
    return o


# speedup vs baseline: 1.0224x; 1.0224x over previous
import functools

import jax
import jax.numpy as jnp
from jax import lax
from jax.experimental import pallas as pl
from jax.experimental.pallas import tpu as pltpu
from jax.experimental.pallas import tpu_sc as plsc

N = 10000
E = 320000
H = 128
EH = 256
G = 128

NC = 2
NS = 16
NW = NC * NS
EW = E // NW
CH = 40
NCH = EW // CH
NG = 5
CPG = NCH // NG
NB = 3
ZS = 10
ZRW = N // ZS

TN = 2000
TE = 4000

_f32 = jnp.float32


def _dot(a, b):
    return jnp.dot(a, b, preferred_element_type=_f32)


def _dot_hi(a, b):
    return jnp.dot(a, b, preferred_element_type=_f32,
                   precision=lax.Precision.HIGHEST)


_BN_DEN = 1.0000050067901611


def _bnk(x, g, be):
    return g * x / _BN_DEN + be


def _onehot(bt):
    iota = lax.broadcasted_iota(jnp.int32, (bt.shape[0], G), 1)
    return jnp.where(bt == iota, 1.0, 0.0).astype(_f32)


def _in_body(x_ref, w_ref, b_ref, v_ref, o_ref):
    h = jnp.maximum(_dot(x_ref[...], w_ref[...]) + b_ref[...], 0.0)
    o_ref[...] = h + v_ref[...]


def _input_layer(x, atom_w, atom_b, vn_row):
    return pl.pallas_call(
        _in_body,
        grid=(N // TN,),
        in_specs=[
            pl.BlockSpec((TN, H), lambda i: (i, 0)),
            pl.BlockSpec((H, H), lambda i: (0, 0)),
            pl.BlockSpec((1, H), lambda i: (0, 0)),
            pl.BlockSpec((1, H), lambda i: (0, 0)),
        ],
        out_specs=pl.BlockSpec((TN, H), lambda i: (i, 0)),
        out_shape=jax.ShapeDtypeStruct((N, H), _f32),
    )(x, atom_w, atom_b, vn_row)


def _edge_body(ea_ref, w1, b1, w2, b2, o_ref):
    t = jnp.maximum(_dot(ea_ref[...], w1[...]) + b1[...], 0.0)
    o_ref[...] = _dot(t, w2[...]) + b2[...]


def _edge_mlp(edge_attr, c):
    de = edge_attr.shape[1]
    return pl.pallas_call(
        _edge_body,
        grid=(E // TE,),
        in_specs=[
            pl.BlockSpec((TE, de), lambda i: (i, 0)),
            pl.BlockSpec((de, EH), lambda i: (0, 0)),
            pl.BlockSpec((1, EH), lambda i: (0, 0)),
            pl.BlockSpec((EH, H), lambda i: (0, 0)),
            pl.BlockSpec((1, H), lambda i: (0, 0)),
        ],
        out_specs=pl.BlockSpec((TE, H), lambda i: (i, 0)),
        out_shape=jax.ShapeDtypeStruct((E, H), _f32),
    )(edge_attr, c["eW1"], c["eb1"].reshape(1, EH), c["eW2"],
      c["eb2"].reshape(1, H))


def _vn_body(h_ref, bf_ref, vn_ref, w1, b1, g1, be1, w2, b2, g2, be2,
             o_ref, acc):
    i = pl.program_id(0)

    @pl.when(i == 0)
    def _():
        acc[...] = jnp.zeros_like(acc)

    oh = _onehot(bf_ref[...])
    acc[...] += lax.dot_general(oh, h_ref[...], (((0,), (0,)), ((), ())),
                                preferred_element_type=_f32,
                                precision=lax.Precision.HIGHEST)

    @pl.when(i == pl.num_programs(0) - 1)
    def _():
        vt = acc[...] + vn_ref[...]
        t2 = jnp.maximum(_bnk(_dot(vt, w1[...]) + b1[...], g1[...], be1[...]),
                         0.0)
        o_ref[...] = jnp.maximum(
            _bnk(_dot(t2, w2[...]) + b2[...], g2[...], be2[...]), 0.0)


def _vn_update(h_cur, batch_f, vn, v):
    return pl.pallas_call(
        _vn_body,
        grid=(N // TN,),
        in_specs=[
            pl.BlockSpec((TN, H), lambda i: (i, 0)),
            pl.BlockSpec((TN, 1), lambda i: (i, 0)),
            pl.BlockSpec((G, H), lambda i: (0, 0)),
            pl.BlockSpec((H, EH), lambda i: (0, 0)),
            pl.BlockSpec((1, EH), lambda i: (0, 0)),
            pl.BlockSpec((1, EH), lambda i: (0, 0)),
            pl.BlockSpec((1, EH), lambda i: (0, 0)),
            pl.BlockSpec((EH, H), lambda i: (0, 0)),
            pl.BlockSpec((1, H), lambda i: (0, 0)),
            pl.BlockSpec((1, H), lambda i: (0, 0)),
            pl.BlockSpec((1, H), lambda i: (0, 0)),
        ],
        out_specs=pl.BlockSpec((G, H), lambda i: (0, 0)),
        out_shape=jax.ShapeDtypeStruct((G, H), _f32),
        scratch_shapes=[pltpu.VMEM((G, H), _f32)],
    )(h_cur, batch_f, vn,
      v["W1"], v["b1"].reshape(1, EH), v["g1"].reshape(1, EH),
      v["be1"].reshape(1, EH),
      v["W2"], v["b2"].reshape(1, H), v["g2"].reshape(1, H),
      v["be2"].reshape(1, H))


@functools.lru_cache(maxsize=None)
def _build_sc_aggregate():
    mesh = plsc.VectorSubcoreMesh(core_axis_name="c", subcore_axis_name="s",
                                  num_cores=NC, num_subcores=NS)
    return functools.partial(
        pl.kernel,
        out_type=jax.ShapeDtypeStruct((2 * N, H), _f32),
        mesh=mesh,
        scratch_types=[
            pltpu.VMEM((CPG, CH), jnp.int32),
            pltpu.VMEM((CPG, CH), jnp.int32),
            [pltpu.VMEM((CH, H), _f32)] * NB,
            [pltpu.VMEM((CH, H), _f32)] * NB,
            pltpu.VMEM_SHARED((N, H), _f32),
            [pltpu.SemaphoreType.DMA] * NB,
            [pltpu.SemaphoreType.DMA] * NB,
        ],
    )(_sc_body)


def _sc_body(h_hbm, ee_hbm, src_hbm, dst_hbm, z_hbm, out_hbm,
             src_v, dst_v, ee_b, hr_b, acc_sh, sem_l, sem_s):
    c = lax.axis_index("c")
    s = lax.axis_index("s")
    wid = c * NS + s

    @pl.when(s < ZS)
    def _():
        pltpu.sync_copy(z_hbm, acc_sh.at[pl.ds(s * ZRW, ZRW)])

    plsc.subcore_barrier()

    base0 = wid * EW

    def issue_loads(g, i, k):
        base = base0 + g * CPG * CH + i * CH
        pltpu.async_copy(ee_hbm.at[pl.ds(base, CH)], ee_b[k], sem_l[k])
        pltpu.async_copy(h_hbm.at[src_v.at[i]], hr_b[k], sem_l[k])

    def wait_loads(g, i, k):
        base = base0 + g * CPG * CH + i * CH
        pltpu.make_async_copy(ee_hbm.at[pl.ds(base, CH)], ee_b[k],
                              sem_l[k]).wait()
        pltpu.make_async_copy(h_hbm.at[src_v.at[i]], hr_b[k],
                              sem_l[k]).wait()

    def wait_scatter(i, k):
        pltpu.make_async_copy(hr_b[k], acc_sh.at[dst_v.at[i]],
                              sem_s[k]).wait()

    def slot(g, i, k, guard=True, issue=True):
        wait_loads(g, i, k)
        kk = (k + 1) % NB

        if guard:
            @pl.when(i >= 2)
            def _():
                wait_scatter(i - 2, kk)
        else:
            wait_scatter(i - 2, kk)

        if issue:
            issue_loads(g, i + 1, kk)

        @plsc.parallel_loop(0, CH, unroll=4)
        def _(r):
            for j in range(H // 16):
                sl = pl.ds(j * 16, 16)
                hr_b[k][r, sl] = jnp.maximum(
                    hr_b[k][r, sl] + ee_b[k][r, sl], 0.0)

        pltpu.async_copy(hr_b[k], acc_sh.at[dst_v.at[i]], sem_s[k], add=True)

    def group(g, carry):
        pltpu.sync_copy(src_hbm.at[wid, g], src_v)
        pltpu.sync_copy(dst_hbm.at[wid, g], dst_v)
        issue_loads(g, 0, 0)

        def macro(m, carry1):
            for j in range(NB):
                slot(g, m * NB + j, j)
            return carry1

        lax.fori_loop(0, (CPG - 2) // NB, macro, 0)
        slot(g, CPG - 2, (CPG - 2) % NB, guard=False, issue=True)
        slot(g, CPG - 1, (CPG - 1) % NB, guard=False, issue=False)
        wait_scatter(CPG - 2, (CPG - 2) % NB)
        wait_scatter(CPG - 1, (CPG - 1) % NB)
        return carry

    lax.fori_loop(0, NG, group, 0)
    plsc.subcore_barrier()

    @pl.when(s < ZS)
    def _():
        pltpu.sync_copy(acc_sh.at[pl.ds(s * ZRW, ZRW)],
                        out_hbm.at[pl.ds(c * N + s * ZRW, ZRW)])


def _edge_aggregate(h_cur, ee, src2, dst2, zrows):
    return _build_sc_aggregate()(h_cur, ee, src2, dst2, zrows)


def _tail_body(h_ref, a0_ref, a1_ref, bf_ref, vnn_ref, w1, b1, g1, be1,
               w2, b2, ep_ref, o_ref):
    t = ep_ref[0, 0] * h_ref[...] + a0_ref[...] + a1_ref[...]
    u = jnp.maximum(_bnk(_dot(t, w1[...]) + b1[...], g1[...], be1[...]), 0.0)
    hn = jnp.maximum(_dot(u, w2[...]) + b2[...], 0.0)
    o_ref[...] = hn + _dot_hi(_onehot(bf_ref[...]), vnn_ref[...])


def _tail(h_cur, aggr2, batch_f, vn_next, c, ep):
    nt = N // TN
    return pl.pallas_call(
        _tail_body,
        grid=(nt,),
        in_specs=[
            pl.BlockSpec((TN, H), lambda i: (i, 0)),
            pl.BlockSpec((TN, H), lambda i: (i, 0)),
            pl.BlockSpec((TN, H), lambda i, nt=nt: (i + nt, 0)),
            pl.BlockSpec((TN, 1), lambda i: (i, 0)),
            pl.BlockSpec((G, H), lambda i: (0, 0)),
            pl.BlockSpec((H, EH), lambda i: (0, 0)),
            pl.BlockSpec((1, EH), lambda i: (0, 0)),
            pl.BlockSpec((1, EH), lambda i: (0, 0)),
            pl.BlockSpec((1, EH), lambda i: (0, 0)),
            pl.BlockSpec((EH, H), lambda i: (0, 0)),
            pl.BlockSpec((1, H), lambda i: (0, 0)),
            pl.BlockSpec((1, 1), lambda i: (0, 0)),
        ],
        out_specs=pl.BlockSpec((TN, H), lambda i: (i, 0)),
        out_shape=jax.ShapeDtypeStruct((N, H), _f32),
    )(h_cur, aggr2, aggr2, batch_f, vn_next,
      c["mW1"], c["mb1"].reshape(1, EH), c["mg1"].reshape(1, EH),
      c["mbe1"].reshape(1, EH),
      c["mW2"], c["mb2"].reshape(1, H), ep)


def _final_body(h_ref, a0_ref, a1_ref, bf_ref, w1, b1, g1, be1, w2, b2,
                ep_ref, l1, c1, l2, c2, l3, c3, l4, c4, o_ref, pool, cnt):
    i = pl.program_id(0)

    @pl.when(i == 0)
    def _():
        pool[...] = jnp.zeros_like(pool)
        cnt[...] = jnp.zeros_like(cnt)

    t = ep_ref[0, 0] * h_ref[...] + a0_ref[...] + a1_ref[...]
    u = jnp.maximum(_bnk(_dot(t, w1[...]) + b1[...], g1[...], be1[...]), 0.0)
    h3 = _dot(u, w2[...]) + b2[...]
    oh = _onehot(bf_ref[...])
    dn = (((0,), (0,)), ((), ()))
    hp = lax.Precision.HIGHEST
    pool[...] += lax.dot_general(oh, h3, dn, preferred_element_type=_f32,
                                 precision=hp)
    cnt[...] += lax.dot_general(oh, jnp.ones((oh.shape[0], 1), _f32), dn,
                                preferred_element_type=_f32, precision=hp)

    @pl.when(i == pl.num_programs(0) - 1)
    def _():
        hg = pool[...] / jnp.maximum(cnt[...], 1.0)
        o = _dot(hg, l1[...]) + c1[...]
        o = _dot(o, l2[...]) + c2[...]
        o = _dot(o, l3[...]) + c3[...]
        o_ref[...] = _dot(o, l4[...]) + c4[...]


def _final(h_cur, aggr2, batch_f, c, ep, lin):
    nt = N // TN
    return pl.pallas_call(
        _final_body,
        grid=(nt,),
        in_specs=[
            pl.BlockSpec((TN, H), lambda i: (i, 0)),
            pl.BlockSpec((TN, H), lambda i: (i, 0)),
            pl.BlockSpec((TN, H), lambda i, nt=nt: (i + nt, 0)),
            pl.BlockSpec((TN, 1), lambda i: (i, 0)),
            pl.BlockSpec((H, EH), lambda i: (0, 0)),
            pl.BlockSpec((1, EH), lambda i: (0, 0)),
            pl.BlockSpec((1, EH), lambda i: (0, 0)),
            pl.BlockSpec((1, EH), lambda i: (0, 0)),
            pl.BlockSpec((EH, H), lambda i: (0, 0)),
            pl.BlockSpec((1, H), lambda i: (0, 0)),
            pl.BlockSpec((1, 1), lambda i: (0, 0)),
            pl.BlockSpec((H, H), lambda i: (0, 0)),
            pl.BlockSpec((1, H), lambda i: (0, 0)),
            pl.BlockSpec((H, H), lambda i: (0, 0)),
            pl.BlockSpec((1, H), lambda i: (0, 0)),
            pl.BlockSpec((H, H), lambda i: (0, 0)),
            pl.BlockSpec((1, H), lambda i: (0, 0)),
            pl.BlockSpec((H, 1), lambda i: (0, 0)),
            pl.BlockSpec((1, 1), lambda i: (0, 0)),
        ],
        out_specs=pl.BlockSpec((G, 1), lambda i: (0, 0)),
        out_shape=jax.ShapeDtypeStruct((G, 1), _f32),
        scratch_shapes=[pltpu.VMEM((G, H), _f32), pltpu.VMEM((G, 1), _f32)],
    )(h_cur, aggr2, aggr2, batch_f,
      c["mW1"], c["mb1"].reshape(1, EH), c["mg1"].reshape(1, EH),
      c["mbe1"].reshape(1, EH),
      c["mW2"], c["mb2"].reshape(1, H), ep,
      lin["W1"], lin["b1"].reshape(1, H),
      lin["W2"], lin["b2"].reshape(1, H),
      lin["W3"], lin["b3"].reshape(1, H),
      lin["W4"], lin["b4"].reshape(1, 1))


def kernel(x, edge_index, edge_attr, batch, params):
    batch_f = batch.reshape(N, 1)
    src2 = edge_index[0].reshape(NW, NG, CPG, CH)
    dst2 = edge_index[1].reshape(NW, NG, CPG, CH)
    zrows = jnp.zeros((ZRW, H), _f32)

    vn_row = params["vn_emb"]
    h_cur = _input_layer(x, params["atom_W"], params["atom_b"].reshape(1, H),
                         vn_row)

    ee = _edge_mlp(edge_attr, params["convs"][0])

    vn = jnp.broadcast_to(vn_row, (G, H))
    for l in range(3):
        c = params["convs"][l]
        ep = (1.0 + c["eps"]).reshape(1, 1)

        aggr2 = _edge_aggregate(h_cur, ee, src2, dst2, zrows)

        if l < 2:
            ee = _edge_mlp(edge_attr, params["convs"][l + 1])
            vn = _vn_update(h_cur, batch_f, vn, params["vns"][l])
            h_cur = _tail(h_cur, aggr2, batch_f, vn, c, ep)
        else:
            o = _final(h_cur, aggr2, batch_f, c, ep, params["lin"])
    return o
